# Initial kernel scaffold; baseline (speedup 1.0000x reference)
import jax
import jax.numpy as jnp
from jax.experimental import pallas as pl


def kernel(feat, modality_id, gamma, beta):
    raise NotImplementedError("write your pallas kernel here")


# speedup vs baseline: 3.8492x; 3.8492x over previous
import jax
import jax.numpy as jnp
from jax.experimental import pallas as pl
from jax.experimental.pallas import tpu as pltpu

DIM_ = 4096
BM_ = 512


def _affine_body(idx_ref, feat_ref, g_ref, b_ref, out_ref):
    out_ref[...] = feat_ref[...] * g_ref[0] + b_ref[0]


def kernel(feat, modality_id, gamma, beta):
    B, D = feat.shape
    M = gamma.shape[0]
    idx = jnp.asarray(modality_id, jnp.int32).reshape(1)
    gamma3 = gamma.reshape(M, 1, D)
    beta3 = beta.reshape(M, 1, D)
    grid = (B // BM_,)
    grid_spec = pltpu.PrefetchScalarGridSpec(
        num_scalar_prefetch=1,
        grid=grid,
        in_specs=[
            pl.BlockSpec((BM_, D), lambda i, idx_ref: (i, 0)),
            pl.BlockSpec((1, 1, D), lambda i, idx_ref: (idx_ref[0], 0, 0)),
            pl.BlockSpec((1, 1, D), lambda i, idx_ref: (idx_ref[0], 0, 0)),
        ],
        out_specs=pl.BlockSpec((BM_, D), lambda i, idx_ref: (i, 0)),
    )
    return pl.pallas_call(
        _affine_body,
        grid_spec=grid_spec,
        out_shape=jax.ShapeDtypeStruct((B, D), feat.dtype),
        compiler_params=pltpu.CompilerParams(
            dimension_semantics=("arbitrary",),
        ),
    )(idx, feat, gamma3, beta3)
